# baseline (speedup 1.0000x reference)
import functools

import jax
import jax.numpy as jnp
from jax import lax
from jax.experimental import pallas as pl
from jax.experimental.pallas import tpu as pltpu
from jax.experimental.pallas import tpu_sc as plsc

B = 16384
F = 26
D = 16
N = B * F
TROWS = 2600000
GROWS = TROWS // 8
NC, NS, L = 2, 16, 16
NW = NC * NS
PER_W = N // NW
CHUNK = 416
SUB = 4
WCHUNK = CHUNK * SUB
NSUPER = PER_W // WCHUNK
FIELD_SCALE = 100000

_mesh = plsc.VectorSubcoreMesh(
    core_axis_name="c", subcore_axis_name="s", num_cores=NC, num_subcores=NS
)


@functools.partial(
    pl.kernel,
    out_type=jax.ShapeDtypeStruct((D, N), jnp.float32),
    mesh=_mesh,
    scratch_types=[
        pltpu.VMEM((CHUNK,), jnp.int32),
        pltpu.VMEM((CHUNK,), jnp.int32),
        pltpu.VMEM((CHUNK,), jnp.int32),
        pltpu.VMEM((CHUNK, 128), jnp.float32),
        pltpu.VMEM((D, WCHUNK), jnp.float32),
        pltpu.SemaphoreType.DMA,
    ],
    compiler_params=pltpu.CompilerParams(needs_layout_passes=False),
)
def _embed(x_hbm, t128_hbm, out_hbm, grp_v, sub_v, off_v, rows_v, cmp_v, gsem):
    wid = lax.axis_index("s") * NC + lax.axis_index("c")
    base = wid * PER_W

    def fill(i, carry):
        lanes = lax.iota(jnp.int32, L) + i * L
        off_v[pl.ds(i * L, L)] = lax.rem(lanes, F) * FIELD_SCALE
        return carry

    lax.fori_loop(0, CHUNK // L, fill, 0)

    def super_body(g, carry):
        gb = base + g * WCHUNK

        def chunk_body(k, carry1):
            cb = pl.multiple_of(gb + k * CHUNK, 8)
            pltpu.sync_copy(x_hbm.at[pl.ds(cb, CHUNK)], grp_v)

            def rowcalc(i, carry2):
                s = pl.ds(i * L, L)
                r = grp_v[s] + off_v[s]
                grp_v[s] = lax.shift_right_logical(r, 3)
                sub_v[s] = lax.shift_left(lax.bitwise_and(r, 7), 4)
                return carry2

            lax.fori_loop(0, CHUNK // L, rowcalc, 0)
            pltpu.async_copy(t128_hbm.at[grp_v], rows_v, gsem).wait()

            def extract(i, carry2):
                nloc = lax.iota(jnp.int32, L) + i * L
                colb = sub_v[pl.ds(i * L, L)]

                for d in range(D):
                    vals = plsc.load_gather(rows_v, [nloc, colb + d])
                    cmp_v[d, pl.ds(k * CHUNK + i * L, L)] = vals
                return carry2

            lax.fori_loop(0, CHUNK // L, extract, 0)
            return carry1

        lax.fori_loop(0, SUB, chunk_body, 0)
        cw = pl.multiple_of(gb, 128)
        pltpu.sync_copy(cmp_v, out_hbm.at[:, pl.ds(cw, WCHUNK)])
        return carry

    lax.fori_loop(0, NSUPER, super_body, 0)


def kernel(x, table):
    xf = x.astype(jnp.int32).reshape(N)
    z = jnp.bitwise_and(xf[0], 0).astype(jnp.float32)
    eye = jnp.eye(128, dtype=jnp.float32) * (1.0 + z)
    t128 = jnp.matmul(
        table.reshape(GROWS, 128), eye, precision=jax.lax.Precision.HIGHEST
    )
    outc = _embed(xf, t128)
    return outc.T.reshape(B, F, D)


# speedup vs baseline: 1.2689x; 1.2689x over previous
import functools

import jax
import jax.numpy as jnp
from jax import lax
from jax.experimental import pallas as pl
from jax.experimental.pallas import tpu as pltpu
from jax.experimental.pallas import tpu_sc as plsc

B = 16384
F = 26
D = 16
N = B * F
TROWS = 2600000
GROWS = TROWS // 8
NC, NS, L = 2, 16, 16
NW = NC * NS
FIELD_SCALE = 100000

ACH = 1024
NACH = 2539
ATAIL = TROWS - NACH * ACH
ATAIL_W = 11

BW = B // NW
SUBB = 128
NSUB = BW // SUBB
SUBN = SUBB * F
GB = 208
NGB = SUBN // GB

_mesh = plsc.VectorSubcoreMesh(
    core_axis_name="c", subcore_axis_name="s", num_cores=NC, num_subcores=NS
)


@functools.partial(
    pl.kernel,
    out_type=jax.ShapeDtypeStruct((F, D, B), jnp.float32),
    mesh=_mesh,
    scratch_types=[
        pltpu.HBM((GROWS, 128), jnp.float32),
        pltpu.VMEM((D, ATAIL), jnp.float32),
        pltpu.VMEM((D, ACH), jnp.float32),
        pltpu.VMEM((ACH // 8, 128), jnp.float32),
        pltpu.VMEM((F, SUBB), jnp.int32),
        pltpu.VMEM((SUBN,), jnp.int32),
        pltpu.VMEM((SUBN,), jnp.int32),
        pltpu.VMEM((GB, 128), jnp.float32),
        pltpu.VMEM((F, D, SUBB), jnp.float32),
        pltpu.SemaphoreType.DMA,
        pltpu.SemaphoreType.REGULAR,
    ],
    compiler_params=pltpu.CompilerParams(needs_layout_passes=False),
)
def _embed(xt_hbm, tt_hbm, tl_hbm, out_hbm, t128_hbm, tlin_v, tin_v, tout_v,
           xv, grp_v, sub_v, rows_v, stage_v, gsem, bsem):
    cid = lax.axis_index("c")
    sid = lax.axis_index("s")
    wid = sid * NC + cid
    iota = lax.iota(jnp.int32, L)

    def transpose_cols(src_v, ncols, g0):
        def col_group(g, carry):
            for u in range(8):
                cl = g * 8 + u
                vals = plsc.load_gather(
                    src_v, [iota, jnp.full((L,), cl, jnp.int32)]
                )
                tout_v[lax.div(cl, 8), pl.ds(lax.rem(cl, 8) * L, L)] = vals
            return carry

        lax.fori_loop(0, ncols // 8, col_group, 0)
        pltpu.sync_copy(
            tout_v.at[: ncols // 8, :],
            t128_hbm.at[pl.ds(pl.multiple_of(g0, 8), ncols // 8)],
        )

    nk = 79 + jnp.where(wid < NACH - 79 * NW, 1, 0)

    def a_chunk(k, carry):
        c0 = (wid + k * NW) * ACH
        pltpu.sync_copy(tt_hbm.at[:, pl.ds(c0, ACH)], tin_v)
        transpose_cols(tin_v, ACH, lax.div(c0, 8))
        return carry

    lax.fori_loop(0, nk, a_chunk, 0)

    @pl.when(wid == ATAIL_W)
    def _tail():
        pltpu.sync_copy(tl_hbm, tlin_v)
        transpose_cols(tlin_v, ATAIL, NACH * ACH // 8)

    plsc.subcore_barrier()
    pl.semaphore_signal(bsem, 1, core_index=1 - cid)
    pl.semaphore_wait(bsem, 1)
    plsc.subcore_barrier()

    def sub_window(sw, carry):
        b0 = pl.multiple_of(wid * BW + sw * SUBB, SUBB)
        pltpu.sync_copy(xt_hbm.at[:, pl.ds(b0, SUBB)], xv)

        def idx_body(i, carry1):
            f = lax.div(i, SUBB // L)
            j = lax.rem(i, SUBB // L)
            r = xv[f, pl.ds(j * L, L)] + f * FIELD_SCALE
            p = pl.ds(i * L, L)
            grp_v[p] = lax.shift_right_logical(r, 3)
            sub_v[p] = lax.shift_left(lax.bitwise_and(r, 7), 4)
            return carry1

        lax.fori_loop(0, SUBN // L, idx_body, 0)

        def gb_body(gb, carry1):
            pltpu.async_copy(
                t128_hbm.at[grp_v.at[pl.ds(gb * GB, GB)]], rows_v, gsem
            ).wait()

            def v_body(v, carry2):
                pv = iota + gb * GB + v * L
                fv = lax.shift_right_logical(pv, 7)
                blv = lax.bitwise_and(pv, SUBB - 1)
                colb = sub_v[pl.ds(gb * GB + v * L, L)]
                nloc = iota + v * L
                for d in range(D):
                    vals = plsc.load_gather(rows_v, [nloc, colb + d])
                    plsc.store_scatter(
                        stage_v, [fv, jnp.full((L,), d, jnp.int32), blv], vals
                    )
                return carry2

            lax.fori_loop(0, GB // L, v_body, 0)
            return carry1

        lax.fori_loop(0, NGB, gb_body, 0)

        pltpu.sync_copy(stage_v, out_hbm.at[:, :, pl.ds(b0, SUBB)])
        return carry

    lax.fori_loop(0, NSUB, sub_window, 0)


def kernel(x, table):
    outc = _embed(x.T, table.T, table[NACH * ACH :].T)
    return outc.transpose(2, 0, 1)


# speedup vs baseline: 2.4928x; 1.9645x over previous
import functools

import jax
import jax.numpy as jnp
from jax import lax
from jax.experimental import pallas as pl
from jax.experimental.pallas import tpu as pltpu
from jax.experimental.pallas import tpu_sc as plsc

B = 16384
F = 26
D = 16
N = B * F
TROWS = 2600000
GROWS = TROWS // 8
NC, NS, L = 2, 16, 16
NW = NC * NS
FIELD_SCALE = 100000

ACH = 1024
NACH = 2539
ATAIL = TROWS - NACH * ACH
ATAIL_W = 11

BW = B // NW
SUBB = 128
NSUB = BW // SUBB
SUBN = SUBB * F
GB = 208
NGB = SUBN // GB

_mesh = plsc.VectorSubcoreMesh(
    core_axis_name="c", subcore_axis_name="s", num_cores=NC, num_subcores=NS
)


@functools.partial(
    pl.kernel,
    out_type=jax.ShapeDtypeStruct((F, D, B), jnp.float32),
    mesh=_mesh,
    scratch_types=[
        pltpu.HBM((GROWS, 128), jnp.float32),
        pltpu.VMEM((D, ATAIL), jnp.float32),
        pltpu.VMEM((D, ACH), jnp.float32),
        pltpu.VMEM((ACH // 8, 128), jnp.float32),
        pltpu.VMEM((F, SUBB), jnp.int32),
        pltpu.VMEM((SUBN,), jnp.int32),
        pltpu.VMEM((SUBN,), jnp.int32),
        pltpu.VMEM((GB, 128), jnp.float32),
        pltpu.VMEM((F, D, SUBB), jnp.float32),
        pltpu.SemaphoreType.DMA,
        pltpu.SemaphoreType.REGULAR,
    ],
    compiler_params=pltpu.CompilerParams(needs_layout_passes=False),
)
def _embed(xt_hbm, tt_hbm, tl_hbm, out_hbm, t128_hbm, tlin_v, tin_v, tout_v,
           xv, grp_v, sub_v, rows_v, stage_v, gsem, bsem):
    cid = lax.axis_index("c")
    sid = lax.axis_index("s")
    wid = sid * NC + cid
    iota = lax.iota(jnp.int32, L)

    def transpose_cols(src_v, ncols, g0):
        def col_group(g, carry):
            cv = iota + g * L
            rowv = lax.shift_right_logical(cv, 3)
            colb = lax.shift_left(lax.bitwise_and(cv, 7), 4)
            for d in range(D):
                vals = src_v[d, pl.ds(g * L, L)]
                plsc.store_scatter(tout_v, [rowv, colb + d], vals)
            return carry

        lax.fori_loop(0, ncols // L, col_group, 0)
        pltpu.sync_copy(
            tout_v.at[: ncols // 8, :],
            t128_hbm.at[pl.ds(pl.multiple_of(g0, 8), ncols // 8)],
        )

    nk = 79 + jnp.where(wid < NACH - 79 * NW, 1, 0)

    def a_chunk(k, carry):
        c0 = (wid + k * NW) * ACH
        pltpu.sync_copy(tt_hbm.at[:, pl.ds(c0, ACH)], tin_v)
        transpose_cols(tin_v, ACH, lax.div(c0, 8))
        return carry

    lax.fori_loop(0, nk, a_chunk, 0)

    @pl.when(wid == ATAIL_W)
    def _tail():
        pltpu.sync_copy(tl_hbm, tlin_v)
        transpose_cols(tlin_v, ATAIL, NACH * ACH // 8)

    plsc.subcore_barrier()
    pl.semaphore_signal(bsem, 1, core_index=1 - cid)
    pl.semaphore_wait(bsem, 1)
    plsc.subcore_barrier()

    def sub_window(sw, carry):
        b0 = pl.multiple_of(wid * BW + sw * SUBB, SUBB)
        pltpu.sync_copy(xt_hbm.at[:, pl.ds(b0, SUBB)], xv)

        def idx_body(i, carry1):
            f = lax.div(i, SUBB // L)
            j = lax.rem(i, SUBB // L)
            r = xv[f, pl.ds(j * L, L)] + f * FIELD_SCALE
            p = pl.ds(i * L, L)
            grp_v[p] = lax.shift_right_logical(r, 3)
            sub_v[p] = lax.shift_left(lax.bitwise_and(r, 7), 4)
            return carry1

        lax.fori_loop(0, SUBN // L, idx_body, 0)

        def gb_body(gb, carry1):
            pltpu.async_copy(
                t128_hbm.at[grp_v.at[pl.ds(gb * GB, GB)]], rows_v, gsem
            ).wait()

            def v_body(v, carry2):
                pv = iota + gb * GB + v * L
                fv = lax.shift_right_logical(pv, 7)
                blv = lax.bitwise_and(pv, SUBB - 1)
                colb = sub_v[pl.ds(gb * GB + v * L, L)]
                nloc = iota + v * L
                for d in range(D):
                    vals = plsc.load_gather(rows_v, [nloc, colb + d])
                    plsc.store_scatter(
                        stage_v, [fv, jnp.full((L,), d, jnp.int32), blv], vals
                    )
                return carry2

            lax.fori_loop(0, GB // L, v_body, 0)
            return carry1

        lax.fori_loop(0, NGB, gb_body, 0)

        pltpu.sync_copy(stage_v, out_hbm.at[:, :, pl.ds(b0, SUBB)])
        return carry

    lax.fori_loop(0, NSUB, sub_window, 0)


def kernel(x, table):
    outc = _embed(x.T, table.T, table[NACH * ACH :].T)
    return outc.transpose(2, 0, 1)


# speedup vs baseline: 2.5713x; 1.0315x over previous
import functools

import jax
import jax.numpy as jnp
from jax import lax
from jax.experimental import pallas as pl
from jax.experimental.pallas import tpu as pltpu
from jax.experimental.pallas import tpu_sc as plsc

B = 16384
F = 26
D = 16
N = B * F
TROWS = 2600000
GROWS = TROWS // 8
NC, NS, L = 2, 16, 16
NW = NC * NS
FIELD_SCALE = 100000

ACH = 512
NACH = 5078
ATAIL = TROWS - NACH * ACH
ATAIL_W = NACH % NW

BW = B // NW
SUBB = 128
NSUB = BW // SUBB
SUBN = SUBB * F
GB = 128
NGB = SUBN // GB

_mesh = plsc.VectorSubcoreMesh(
    core_axis_name="c", subcore_axis_name="s", num_cores=NC, num_subcores=NS
)


@functools.partial(
    pl.kernel,
    out_type=jax.ShapeDtypeStruct((F, D, B), jnp.float32),
    mesh=_mesh,
    scratch_types=[
        pltpu.HBM((GROWS, 128), jnp.float32),
        pltpu.VMEM((D, ATAIL), jnp.float32),
        pltpu.VMEM((D, ACH), jnp.float32),
        pltpu.VMEM((ACH // 8, 128), jnp.float32),
        pltpu.VMEM((F, SUBB), jnp.int32),
        pltpu.VMEM((SUBN,), jnp.int32),
        pltpu.VMEM((SUBN,), jnp.int32),
        pltpu.VMEM((GB, 128), jnp.float32),
        pltpu.VMEM((GB, 128), jnp.float32),
        pltpu.VMEM((F, D, SUBB), jnp.float32),
        pltpu.SemaphoreType.DMA,
        pltpu.SemaphoreType.DMA,
        pltpu.SemaphoreType.REGULAR,
    ],
    compiler_params=pltpu.CompilerParams(needs_layout_passes=False),
)
def _embed(xt_hbm, tt_hbm, tl_hbm, out_hbm, t128_hbm, tlin_v, tin_v, tout_v,
           xv, grp_v, sub_v, rows_a, rows_b, stage_v, gsem_a, gsem_b, bsem):
    cid = lax.axis_index("c")
    sid = lax.axis_index("s")
    wid = sid * NC + cid
    iota = lax.iota(jnp.int32, L)

    def transpose_cols(src_v, ncols, g0):
        def col_group(g, carry):
            cv = iota + g * L
            rowv = lax.shift_right_logical(cv, 3)
            colb = lax.shift_left(lax.bitwise_and(cv, 7), 4)
            for d in range(D):
                vals = src_v[d, pl.ds(g * L, L)]
                plsc.store_scatter(tout_v, [rowv, colb + d], vals)
            return carry

        lax.fori_loop(0, ncols // L, col_group, 0)
        pltpu.sync_copy(
            tout_v.at[: ncols // 8, :],
            t128_hbm.at[pl.ds(pl.multiple_of(g0, 8), ncols // 8)],
        )

    nkf = NACH // NW
    nk = nkf + jnp.where(wid < NACH - nkf * NW, 1, 0)

    def a_chunk(k, carry):
        c0 = (wid + k * NW) * ACH
        pltpu.sync_copy(tt_hbm.at[:, pl.ds(c0, ACH)], tin_v)
        transpose_cols(tin_v, ACH, lax.div(c0, 8))
        return carry

    lax.fori_loop(0, nk, a_chunk, 0)

    @pl.when(wid == ATAIL_W)
    def _tail():
        pltpu.sync_copy(tl_hbm, tlin_v)
        transpose_cols(tlin_v, ATAIL, NACH * ACH // 8)

    plsc.subcore_barrier()
    pl.semaphore_signal(bsem, 1, core_index=1 - cid)
    pl.semaphore_wait(bsem, 1)
    plsc.subcore_barrier()

    def sub_window(sw, carry):
        b0 = pl.multiple_of(wid * BW + sw * SUBB, SUBB)
        pltpu.sync_copy(xt_hbm.at[:, pl.ds(b0, SUBB)], xv)

        def idx_body(i, carry1):
            f = lax.div(i, SUBB // L)
            j = lax.rem(i, SUBB // L)
            r = xv[f, pl.ds(j * L, L)] + f * FIELD_SCALE
            p = pl.ds(i * L, L)
            grp_v[p] = lax.shift_right_logical(r, 3)
            sub_v[p] = lax.shift_left(lax.bitwise_and(r, 7), 4)
            return carry1

        lax.fori_loop(0, SUBN // L, idx_body, 0)

        bufs = (rows_a, rows_b)
        sems = (gsem_a, gsem_b)

        def start(gb, buf, sem):
            return pltpu.async_copy(
                t128_hbm.at[grp_v.at[pl.ds(gb * GB, GB)]], buf, sem
            )

        def extract(gb, buf):
            def v_body(v, carry2):
                pv = iota + gb * GB + v * L
                fv = lax.shift_right_logical(pv, 7)
                blv = lax.bitwise_and(pv, SUBB - 1)
                colb = sub_v[pl.ds(gb * GB + v * L, L)]
                nloc = iota + v * L
                for d in range(D):
                    vals = plsc.load_gather(buf, [nloc, colb + d])
                    plsc.store_scatter(
                        stage_v, [fv, jnp.full((L,), d, jnp.int32), blv], vals
                    )
                return carry2

            lax.fori_loop(0, GB // L, v_body, 0)

        cps = [start(0, bufs[0], sems[0]), None]
        for gb in range(NGB):
            cur, nxt = gb % 2, (gb + 1) % 2
            if gb + 1 < NGB:
                cps[nxt] = start(gb + 1, bufs[nxt], sems[nxt])
            cps[cur].wait()
            extract(gb, bufs[cur])

        pltpu.sync_copy(stage_v, out_hbm.at[:, :, pl.ds(b0, SUBB)])
        return carry

    lax.fori_loop(0, NSUB, sub_window, 0)


def kernel(x, table):
    outc = _embed(x.T, table.T, table[NACH * ACH :].T)
    return outc.transpose(2, 0, 1)


# speedup vs baseline: 3.7131x; 1.4440x over previous
import functools

import jax
import jax.numpy as jnp
from jax import lax
from jax.experimental import pallas as pl
from jax.experimental.pallas import tpu as pltpu
from jax.experimental.pallas import tpu_sc as plsc

B = 16384
F = 26
D = 16
N = B * F
TROWS = 2600000
GROWS = TROWS // 8
NC, NS, L = 2, 16, 16
NW = NC * NS
FIELD_SCALE = 100000

ACH = 256
NACH = 10156
ATAIL = TROWS - NACH * ACH
ATAIL_W = NACH % NW

BW = B // NW
SUBB = 128
NSUB = BW // SUBB
SUBN = SUBB * F
GB = 128
NGB = SUBN // GB

_mesh = plsc.VectorSubcoreMesh(
    core_axis_name="c", subcore_axis_name="s", num_cores=NC, num_subcores=NS
)


@functools.partial(
    pl.kernel,
    out_type=jax.ShapeDtypeStruct((F, D, B), jnp.float32),
    mesh=_mesh,
    scratch_types=[
        pltpu.HBM((GROWS, 128), jnp.float32),
        pltpu.VMEM((D, ATAIL), jnp.float32),
        pltpu.VMEM((D, ACH), jnp.float32),
        pltpu.VMEM((D, ACH), jnp.float32),
        pltpu.VMEM((ACH // 8, 128), jnp.float32),
        pltpu.VMEM((ACH // 8, 128), jnp.float32),
        pltpu.SemaphoreType.DMA,
        pltpu.SemaphoreType.DMA,
        pltpu.SemaphoreType.DMA,
        pltpu.SemaphoreType.DMA,
        pltpu.VMEM((F, SUBB), jnp.int32),
        pltpu.VMEM((SUBN,), jnp.int32),
        pltpu.VMEM((SUBN,), jnp.int32),
        pltpu.VMEM((GB, 128), jnp.float32),
        pltpu.VMEM((GB, 128), jnp.float32),
        pltpu.VMEM((F, D, SUBB), jnp.float32),
        pltpu.SemaphoreType.DMA,
        pltpu.SemaphoreType.DMA,
        pltpu.SemaphoreType.REGULAR,
    ],
    compiler_params=pltpu.CompilerParams(needs_layout_passes=False),
)
def _embed(xt_hbm, tt_hbm, tl_hbm, out_hbm, t128_hbm, tlin_v, tin_a, tin_b,
           tout_a, tout_b, isem_a, isem_b, osem_a, osem_b, xv, grp_v, sub_v,
           rows_a, rows_b, stage_v, gsem_a, gsem_b, bsem):
    cid = lax.axis_index("c")
    sid = lax.axis_index("s")
    wid = sid * NC + cid
    iota = lax.iota(jnp.int32, L)

    def transpose_cols(src_v, dst_v, ncols):
        def col_group(g, carry):
            cv = iota + g * L
            rowv = lax.shift_right_logical(cv, 3)
            colb = lax.shift_left(lax.bitwise_and(cv, 7), 4)
            for d in range(D):
                vals = src_v[d, pl.ds(g * L, L)]
                plsc.store_scatter(dst_v, [rowv, colb + d], vals)
            return carry

        lax.fori_loop(0, ncols // L, col_group, 0)

    nkf = NACH // NW
    nk = nkf + jnp.where(wid < NACH - nkf * NW, 1, 0)

    def a_c0(k):
        return (wid + k * NW) * ACH

    def a_in(k, tin, isem):
        return pltpu.async_copy(tt_hbm.at[:, pl.ds(a_c0(k), ACH)], tin, isem)

    def a_out(k, tout, osem):
        g0 = pl.multiple_of(lax.div(a_c0(k), 8), 8)
        return pltpu.async_copy(tout, t128_hbm.at[pl.ds(g0, ACH // 8)], osem)

    a_in(0, tin_a, isem_a)

    @pl.when(1 < nk)
    def _primeb():
        a_in(1, tin_b, isem_b)

    def a_pair(j, carry):
        for kk, tin, tout, isem, osem in (
            (2 * j, tin_a, tout_a, isem_a, osem_a),
            (2 * j + 1, tin_b, tout_b, isem_b, osem_b),
        ):
            @pl.when(kk < nk)
            def _do():
                pltpu.make_async_copy(
                    tt_hbm.at[:, pl.ds(a_c0(kk), ACH)], tin, isem
                ).wait()

                @pl.when(kk >= 2)
                def _wout():
                    g0p = pl.multiple_of(lax.div(a_c0(kk - 2), 8), 8)
                    pltpu.make_async_copy(
                        tout, t128_hbm.at[pl.ds(g0p, ACH // 8)], osem
                    ).wait()

                transpose_cols(tin, tout, ACH)
                a_out(kk, tout, osem)

                @pl.when(kk + 2 < nk)
                def _nin():
                    a_in(kk + 2, tin, isem)

        return carry

    lax.fori_loop(0, (nk + 1) // 2, a_pair, 0)

    for tout, osem, par in ((tout_a, osem_a, 0), (tout_b, osem_b, 1)):
        @pl.when(nk >= par + 1)
        def _drain():
            kl = nk - 1
            kl = jnp.where(lax.rem(kl, 2) == par, kl, kl - 1)
            g0l = pl.multiple_of(lax.div(a_c0(kl), 8), 8)
            pltpu.make_async_copy(
                tout, t128_hbm.at[pl.ds(g0l, ACH // 8)], osem
            ).wait()

    @pl.when(wid == ATAIL_W)
    def _tail():
        pltpu.sync_copy(tl_hbm, tlin_v)
        transpose_cols(tlin_v, tout_a, ATAIL)
        pltpu.sync_copy(
            tout_a.at[: ATAIL // 8, :],
            t128_hbm.at[pl.ds(NACH * ACH // 8, ATAIL // 8)],
        )

    plsc.subcore_barrier()
    pl.semaphore_signal(bsem, 1, core_index=1 - cid)
    pl.semaphore_wait(bsem, 1)
    plsc.subcore_barrier()

    def sub_window(sw, carry):
        b0 = pl.multiple_of(wid * BW + sw * SUBB, SUBB)
        pltpu.sync_copy(xt_hbm.at[:, pl.ds(b0, SUBB)], xv)

        def idx_body(i, carry1):
            f = lax.div(i, SUBB // L)
            j = lax.rem(i, SUBB // L)
            r = xv[f, pl.ds(j * L, L)] + f * FIELD_SCALE
            p = pl.ds(i * L, L)
            grp_v[p] = lax.shift_right_logical(r, 3)
            sub_v[p] = lax.shift_left(lax.bitwise_and(r, 7), 4)
            return carry1

        lax.fori_loop(0, SUBN // L, idx_body, 0)

        bufs = (rows_a, rows_b)
        sems = (gsem_a, gsem_b)

        def start(gb, buf, sem):
            return pltpu.async_copy(
                t128_hbm.at[grp_v.at[pl.ds(gb * GB, GB)]], buf, sem
            )

        def extract(gb, buf):
            def v_body(v, carry2):
                pv = iota + gb * GB + v * L
                fv = lax.shift_right_logical(pv, 7)
                blv = lax.bitwise_and(pv, SUBB - 1)
                colb = sub_v[pl.ds(gb * GB + v * L, L)]
                nloc = iota + v * L
                for d in range(D):
                    vals = plsc.load_gather(buf, [nloc, colb + d])
                    plsc.store_scatter(
                        stage_v, [fv, jnp.full((L,), d, jnp.int32), blv], vals
                    )
                return carry2

            lax.fori_loop(0, GB // L, v_body, 0)

        cps = [start(0, bufs[0], sems[0]), None]
        for gb in range(NGB):
            cur, nxt = gb % 2, (gb + 1) % 2
            if gb + 1 < NGB:
                cps[nxt] = start(gb + 1, bufs[nxt], sems[nxt])
            cps[cur].wait()
            extract(gb, bufs[cur])

        pltpu.sync_copy(stage_v, out_hbm.at[:, :, pl.ds(b0, SUBB)])
        return carry

    lax.fori_loop(0, NSUB, sub_window, 0)


def kernel(x, table):
    outc = _embed(x.T, table.T, table[NACH * ACH :].T)
    return outc.transpose(2, 0, 1)


# speedup vs baseline: 3.7463x; 1.0090x over previous
import functools

import jax
import jax.numpy as jnp
from jax import lax
from jax.experimental import pallas as pl
from jax.experimental.pallas import tpu as pltpu
from jax.experimental.pallas import tpu_sc as plsc

B = 16384
F = 26
D = 16
N = B * F
TROWS = 2600000
GROWS = TROWS // 8
NC, NS, L = 2, 16, 16
NW = NC * NS
FIELD_SCALE = 100000

ACH = 256
NACH = 10156
ATAIL = TROWS - NACH * ACH
ATAIL_W = NACH % NW

BW = B // NW
SUBB = 128
NSUB = BW // SUBB
SUBN = SUBB * F
GB = 128
NGB = SUBN // GB

_mesh = plsc.VectorSubcoreMesh(
    core_axis_name="c", subcore_axis_name="s", num_cores=NC, num_subcores=NS
)


@functools.partial(
    pl.kernel,
    out_type=jax.ShapeDtypeStruct((F, D, B), jnp.float32),
    mesh=_mesh,
    scratch_types=[
        pltpu.HBM((GROWS, 128), jnp.float32),
        pltpu.VMEM((D, ATAIL), jnp.float32),
        pltpu.VMEM((D, ACH), jnp.float32),
        pltpu.VMEM((D, ACH), jnp.float32),
        pltpu.VMEM((ACH // 8, 128), jnp.float32),
        pltpu.VMEM((ACH // 8, 128), jnp.float32),
        pltpu.SemaphoreType.DMA,
        pltpu.SemaphoreType.DMA,
        pltpu.SemaphoreType.DMA,
        pltpu.SemaphoreType.DMA,
        pltpu.VMEM((F, SUBB), jnp.int32),
        pltpu.VMEM((SUBN,), jnp.int32),
        pltpu.VMEM((SUBN,), jnp.int32),
        pltpu.VMEM((GB, 128), jnp.float32),
        pltpu.VMEM((GB, 128), jnp.float32),
        pltpu.VMEM((F, D, SUBB), jnp.float32),
        pltpu.SemaphoreType.DMA,
        pltpu.SemaphoreType.DMA,
        pltpu.SemaphoreType.DMA,
        pltpu.SemaphoreType.REGULAR,
    ],
    compiler_params=pltpu.CompilerParams(needs_layout_passes=False),
)
def _embed(xt_hbm, tt_hbm, tl_hbm, out_hbm, t128_hbm, tlin_v, tin_a, tin_b,
           tout_a, tout_b, isem_a, isem_b, osem_a, osem_b, xv, grp_v, sub_v,
           rows_a, rows_b, stage_v, gsem_a, gsem_b, ssem, bsem):
    cid = lax.axis_index("c")
    sid = lax.axis_index("s")
    wid = sid * NC + cid
    iota = lax.iota(jnp.int32, L)

    def transpose_cols(src_v, dst_v, ncols):
        def col_group(g, carry):
            cv = iota + g * L
            rowv = lax.shift_right_logical(cv, 3)
            colb = lax.shift_left(lax.bitwise_and(cv, 7), 4)
            for d in range(D):
                vals = src_v[d, pl.ds(g * L, L)]
                plsc.store_scatter(dst_v, [rowv, colb + d], vals)
            return carry

        lax.fori_loop(0, ncols // L, col_group, 0)

    nkf = NACH // NW
    nk = nkf + jnp.where(wid < NACH - nkf * NW, 1, 0)

    def a_c0(k):
        return (wid + k * NW) * ACH

    def a_in(k, tin, isem):
        return pltpu.async_copy(tt_hbm.at[:, pl.ds(a_c0(k), ACH)], tin, isem)

    def a_out(k, tout, osem):
        g0 = pl.multiple_of(lax.div(a_c0(k), 8), 8)
        return pltpu.async_copy(tout, t128_hbm.at[pl.ds(g0, ACH // 8)], osem)

    a_in(0, tin_a, isem_a)

    @pl.when(1 < nk)
    def _primeb():
        a_in(1, tin_b, isem_b)

    def a_pair(j, carry):
        for kk, tin, tout, isem, osem in (
            (2 * j, tin_a, tout_a, isem_a, osem_a),
            (2 * j + 1, tin_b, tout_b, isem_b, osem_b),
        ):
            @pl.when(kk < nk)
            def _do():
                pltpu.make_async_copy(
                    tt_hbm.at[:, pl.ds(a_c0(kk), ACH)], tin, isem
                ).wait()

                @pl.when(kk >= 2)
                def _wout():
                    g0p = pl.multiple_of(lax.div(a_c0(kk - 2), 8), 8)
                    pltpu.make_async_copy(
                        tout, t128_hbm.at[pl.ds(g0p, ACH // 8)], osem
                    ).wait()

                transpose_cols(tin, tout, ACH)
                a_out(kk, tout, osem)

                @pl.when(kk + 2 < nk)
                def _nin():
                    a_in(kk + 2, tin, isem)

        return carry

    lax.fori_loop(0, (nk + 1) // 2, a_pair, 0)

    for tout, osem, par in ((tout_a, osem_a, 0), (tout_b, osem_b, 1)):
        @pl.when(nk >= par + 1)
        def _drain():
            kl = nk - 1
            kl = jnp.where(lax.rem(kl, 2) == par, kl, kl - 1)
            g0l = pl.multiple_of(lax.div(a_c0(kl), 8), 8)
            pltpu.make_async_copy(
                tout, t128_hbm.at[pl.ds(g0l, ACH // 8)], osem
            ).wait()

    @pl.when(wid == ATAIL_W)
    def _tail():
        pltpu.sync_copy(tl_hbm, tlin_v)
        transpose_cols(tlin_v, tout_a, ATAIL)
        pltpu.sync_copy(
            tout_a.at[: ATAIL // 8, :],
            t128_hbm.at[pl.ds(NACH * ACH // 8, ATAIL // 8)],
        )

    plsc.subcore_barrier()
    pl.semaphore_signal(bsem, 1, core_index=1 - cid)
    pl.semaphore_wait(bsem, 1)
    plsc.subcore_barrier()

    def sub_window(sw, carry):
        b0 = pl.multiple_of(wid * BW + sw * SUBB, SUBB)
        pltpu.sync_copy(xt_hbm.at[:, pl.ds(b0, SUBB)], xv)

        def idx_body(i, carry1):
            f = lax.div(i, SUBB // L)
            j = lax.rem(i, SUBB // L)
            r = xv[f, pl.ds(j * L, L)] + f * FIELD_SCALE
            p = pl.ds(i * L, L)
            grp_v[p] = lax.shift_right_logical(r, 3)
            sub_v[p] = lax.shift_left(lax.bitwise_and(r, 7), 4)
            return carry1

        lax.fori_loop(0, SUBN // L, idx_body, 0)

        bufs = (rows_a, rows_b)
        sems = (gsem_a, gsem_b)

        def start(gb, buf, sem):
            return pltpu.async_copy(
                t128_hbm.at[grp_v.at[pl.ds(gb * GB, GB)]], buf, sem
            )

        def extract(gb, buf):
            def v_body(v, carry2):
                pv = iota + gb * GB + v * L
                fv = lax.shift_right_logical(pv, 7)
                blv = lax.bitwise_and(pv, SUBB - 1)
                colb = sub_v[pl.ds(gb * GB + v * L, L)]
                nloc = iota + v * L
                for d in range(D):
                    vals = plsc.load_gather(buf, [nloc, colb + d])
                    plsc.store_scatter(
                        stage_v, [fv, jnp.full((L,), d, jnp.int32), blv], vals
                    )
                return carry2

            lax.fori_loop(0, GB // L, v_body, 0)

        cps = [start(0, bufs[0], sems[0]), None]

        @pl.when(sw > 0)
        def _wstage():
            pltpu.make_async_copy(
                stage_v, out_hbm.at[:, :, pl.ds(b0 - SUBB, SUBB)], ssem
            ).wait()

        for gb in range(NGB):
            cur, nxt = gb % 2, (gb + 1) % 2
            if gb + 1 < NGB:
                cps[nxt] = start(gb + 1, bufs[nxt], sems[nxt])
            cps[cur].wait()
            extract(gb, bufs[cur])

        pltpu.async_copy(stage_v, out_hbm.at[:, :, pl.ds(b0, SUBB)], ssem)
        return carry

    lax.fori_loop(0, NSUB, sub_window, 0)
    bl = pl.multiple_of(wid * BW + (NSUB - 1) * SUBB, SUBB)
    pltpu.make_async_copy(
        stage_v, out_hbm.at[:, :, pl.ds(bl, SUBB)], ssem
    ).wait()


def kernel(x, table):
    outc = _embed(x.T, table.T, table[NACH * ACH :].T)
    return outc.transpose(2, 0, 1)
